# Initial kernel scaffold; baseline (speedup 1.0000x reference)
import jax
import jax.numpy as jnp
from jax.experimental import pallas as pl


def kernel(x, router_w, Wg, Wd):
    raise NotImplementedError("write your pallas kernel here")


# speedup vs baseline: 2.2864x; 2.2864x over previous
import functools

import jax
import jax.numpy as jnp
from jax.experimental import pallas as pl
from jax.experimental.pallas import tpu as pltpu

B, S, D = 1, 2048, 768
E, K, F = 16, 2, 384
T = B * S


def _moe_body(x_ref, rw_ref, wg_ref, wd_ref, y_ref, wdense_ref):
    e = pl.program_id(0)
    ids = jax.lax.broadcasted_iota(jnp.int32, (T, E), 1)

    @pl.when(e == 0)
    def _router():
        logits = jax.lax.dot_general(
            x_ref[...], rw_ref[...], (((1,), (1,)), ((), ())),
            preferred_element_type=jnp.float32)
        m = jnp.max(logits, axis=-1, keepdims=True)
        p = jnp.exp(logits - m)
        p = p / jnp.sum(p, axis=-1, keepdims=True)
        m1 = jnp.max(p, axis=-1, keepdims=True)
        i1 = jnp.min(jnp.where(p == m1, ids, E), axis=-1, keepdims=True)
        p2 = jnp.where(ids == i1, -1.0, p)
        m2 = jnp.max(p2, axis=-1, keepdims=True)
        i2 = jnp.min(jnp.where(p2 == m2, ids, E), axis=-1, keepdims=True)
        s = m1 + m2 + 1e-20
        wdense_ref[...] = (jnp.where(ids == i1, m1 / s, 0.0)
                           + jnp.where(ids == i2, m2 / s, 0.0))
        y_ref[...] = jnp.zeros_like(y_ref)

    wg = wg_ref[0]
    h = jax.lax.dot_general(
        x_ref[...], wg, (((1,), (1,)), ((), ())),
        preferred_element_type=jnp.float32)
    gate = h[:, :F]
    proj = h[:, F:]
    a = gate / (1.0 + jnp.exp(-gate)) * proj
    out = jax.lax.dot_general(
        a, wd_ref[0], (((1,), (1,)), ((), ())),
        preferred_element_type=jnp.float32)
    col = jnp.sum(jnp.where(ids == e, wdense_ref[...], 0.0),
                  axis=-1, keepdims=True)
    y_ref[...] += col * out


@jax.jit
def kernel(x, router_w, Wg, Wd):
    hs = x.reshape(T, D)
    y = pl.pallas_call(
        _moe_body,
        grid=(E,),
        in_specs=[
            pl.BlockSpec((T, D), lambda e: (0, 0)),
            pl.BlockSpec((E, D), lambda e: (0, 0)),
            pl.BlockSpec((1, 2 * F, D), lambda e: (e, 0, 0)),
            pl.BlockSpec((1, D, F), lambda e: (e, 0, 0)),
        ],
        out_specs=pl.BlockSpec((T, D), lambda e: (0, 0)),
        out_shape=jax.ShapeDtypeStruct((T, D), jnp.float32),
        scratch_shapes=[pltpu.VMEM((T, E), jnp.float32)],
        compiler_params=pltpu.CompilerParams(
            dimension_semantics=("arbitrary",)),
    )(hs, router_w, Wg, Wd)
    return y.reshape(B, S, D)


# speedup vs baseline: 2.2936x; 1.0031x over previous
import functools

import jax
import jax.numpy as jnp
from jax.experimental import pallas as pl
from jax.experimental.pallas import tpu as pltpu

B, S, D = 1, 2048, 768
E, K, F = 16, 2, 384
T = B * S


def _moe_body(x_ref, rw_ref, wg_ref, wd_ref, y_ref, wdense_ref):
    e = pl.program_id(0)
    ids = jax.lax.broadcasted_iota(jnp.int32, (T, E), 1)

    @pl.when(e == 0)
    def _router():
        logits = jax.lax.dot_general(
            x_ref[...], rw_ref[...], (((1,), (1,)), ((), ())),
            preferred_element_type=jnp.float32)
        m = jnp.max(logits, axis=-1, keepdims=True)
        p = jnp.exp(logits - m)
        p = p / jnp.sum(p, axis=-1, keepdims=True)
        m1 = jnp.max(p, axis=-1, keepdims=True)
        i1 = jnp.min(jnp.where(p == m1, ids, E), axis=-1, keepdims=True)
        p2 = jnp.where(ids == i1, -1.0, p)
        m2 = jnp.max(p2, axis=-1, keepdims=True)
        i2 = jnp.min(jnp.where(p2 == m2, ids, E), axis=-1, keepdims=True)
        s = m1 + m2 + 1e-20
        wdense_ref[...] = (jnp.where(ids == i1, m1 / s, 0.0)
                           + jnp.where(ids == i2, m2 / s, 0.0))
        y_ref[...] = jnp.zeros_like(y_ref)

    wg = wg_ref[0].astype(jnp.bfloat16)
    xb = x_ref[...].astype(jnp.bfloat16)
    h = jax.lax.dot_general(
        xb, wg, (((1,), (1,)), ((), ())),
        preferred_element_type=jnp.float32)
    gate = h[:, :F]
    proj = h[:, F:]
    a = gate / (1.0 + jnp.exp(-gate)) * proj
    out = jax.lax.dot_general(
        a.astype(jnp.bfloat16), wd_ref[0].astype(jnp.bfloat16),
        (((1,), (1,)), ((), ())),
        preferred_element_type=jnp.float32)
    col = jnp.sum(jnp.where(ids == e, wdense_ref[...], 0.0),
                  axis=-1, keepdims=True)
    y_ref[...] += col * out


@jax.jit
def kernel(x, router_w, Wg, Wd):
    hs = x.reshape(T, D)
    y = pl.pallas_call(
        _moe_body,
        grid=(E,),
        in_specs=[
            pl.BlockSpec((T, D), lambda e: (0, 0)),
            pl.BlockSpec((E, D), lambda e: (0, 0)),
            pl.BlockSpec((1, 2 * F, D), lambda e: (e, 0, 0)),
            pl.BlockSpec((1, D, F), lambda e: (e, 0, 0)),
        ],
        out_specs=pl.BlockSpec((T, D), lambda e: (0, 0)),
        out_shape=jax.ShapeDtypeStruct((T, D), jnp.float32),
        scratch_shapes=[pltpu.VMEM((T, E), jnp.float32)],
        compiler_params=pltpu.CompilerParams(
            dimension_semantics=("arbitrary",)),
    )(hs, router_w, Wg, Wd)
    return y.reshape(B, S, D)
